# baseline (speedup 1.0000x reference)
import jax
import jax.numpy as jnp
from jax import lax
from jax.experimental import pallas as pl
from jax.experimental.pallas import tpu as pltpu
import jax.experimental.pallas.tpu_sc as plsc

N_NODES = 10000
E = 160000
D_IN = 16
D_OUT = 32
D_EDGE = 16
HIDDEN = 16

NC = 2
NS = 16
NW = NC * NS
CHUNK = 128
NCHUNK = E // CHUNK
MAXCPW = -(-NCHUNK // NW)
WIDTH = 40
N_PAD = 10240
ROWS_PS = N_PAD // NS


def _sc_mesh():
  return plsc.VectorSubcoreMesh(
      core_axis_name="c", subcore_axis_name="s", num_cores=NC, num_subcores=NS)


def _worker_range(wid):
  c0 = (wid * NCHUNK) // NW
  c1 = ((wid + 1) * NCHUNK) // NW
  return c0, c1


def _gather_body(node_hbm, dst2_hbm, x1_hbm, idx_v, buf_v, sem):
  c = lax.axis_index("c")
  s = lax.axis_index("s")
  wid = s * NC + c
  c0, c1 = _worker_range(wid)
  t = c1 - c0
  pltpu.sync_copy(dst2_hbm.at[pl.ds(c0, MAXCPW)], idx_v)

  def fire(j, _):
    pltpu.async_copy(node_hbm.at[idx_v.at[j]],
                     buf_v.at[pl.ds(j * CHUNK, CHUNK)], sem)
    return _

  lax.fori_loop(0, t, fire, 0, unroll=False)
  base = c0 * CHUNK
  low = (MAXCPW - 1) * CHUNK
  pltpu.make_async_copy(x1_hbm.at[pl.ds(0, low)],
                        buf_v.at[pl.ds(0, low)], sem).wait()

  @pl.when(t == MAXCPW)
  def _():
    pltpu.make_async_copy(x1_hbm.at[pl.ds(0, CHUNK)],
                          buf_v.at[pl.ds(0, CHUNK)], sem).wait()

  pltpu.sync_copy(buf_v.at[pl.ds(0, low)], x1_hbm.at[pl.ds(base, low)])

  @pl.when(t == MAXCPW)
  def _():
    pltpu.sync_copy(buf_v.at[pl.ds(low, CHUNK)],
                    x1_hbm.at[pl.ds(base + low, CHUNK)])


def _gather(node_attr, dst2):
  return pl.kernel(
      _gather_body,
      out_type=jax.ShapeDtypeStruct((E, D_IN), jnp.float32),
      mesh=_sc_mesh(),
      scratch_types=[
          pltpu.VMEM((MAXCPW, CHUNK), jnp.int32),
          pltpu.VMEM((MAXCPW * CHUNK, D_IN), jnp.float32),
          pltpu.SemaphoreType.DMA,
      ],
      compiler_params=pltpu.CompilerParams(use_tc_tiling_on_sc=False),
  )(node_attr, dst2)


def _scatter_body(tpc_hbm, src2_hbm, zeros_hbm, out_hbm, idx_v, val_v, acc_sh):
  c = lax.axis_index("c")
  s = lax.axis_index("s")
  wid = s * NC + c
  c0, c1 = _worker_range(wid)
  pltpu.sync_copy(src2_hbm.at[pl.ds(c0, MAXCPW)], idx_v)
  pltpu.sync_copy(zeros_hbm.at[pl.ds(s * ROWS_PS, ROWS_PS)],
                  acc_sh.at[pl.ds(s * ROWS_PS, ROWS_PS)])
  plsc.subcore_barrier()

  def step(j, _):
    pltpu.sync_copy(tpc_hbm.at[pl.ds((c0 + j) * CHUNK, CHUNK)], val_v)
    pltpu.sync_copy(val_v, acc_sh.at[idx_v.at[j]], add=True)
    return _

  lax.fori_loop(0, c1 - c0, step, 0, unroll=False)
  plsc.subcore_barrier()
  pltpu.sync_copy(acc_sh.at[pl.ds(s * ROWS_PS, ROWS_PS)],
                  out_hbm.at[pl.ds(c * N_PAD + s * ROWS_PS, ROWS_PS)])


def _scatter(tpc, src2, zeros_np):
  return pl.kernel(
      _scatter_body,
      out_type=jax.ShapeDtypeStruct((NC * N_PAD, WIDTH), jnp.float32),
      mesh=_sc_mesh(),
      scratch_types=[
          pltpu.VMEM((MAXCPW, CHUNK), jnp.int32),
          pltpu.VMEM((CHUNK, WIDTH), jnp.float32),
          pltpu.VMEM_SHARED((N_PAD, WIDTH), jnp.float32),
      ],
      compiler_params=pltpu.CompilerParams(use_tc_tiling_on_sc=False),
  )(tpc, src2, zeros_np)


def _mish(x):
  sp = jnp.maximum(x, 0.0) + jnp.log1p(jnp.exp(-jnp.abs(x)))
  return x * jnp.tanh(sp)


BE = 3200


def _dense_body(ea_ref, x1_ref, sh_ref, w1_ref, b1_ref, w2e_ref, out_ref):
  ea = ea_ref[...]
  x1 = x1_ref[...]
  h = _mish(jnp.dot(ea, w1_ref[...], preferred_element_type=jnp.float32)
            + b1_ref[...])
  u = (h[:, :, None] * x1[:, None, :]).reshape(BE, HIDDEN * D_IN)
  ux = jnp.concatenate([u, x1], axis=1)
  tp0 = jnp.dot(ux, w2e_ref[...], preferred_element_type=jnp.float32)
  tp = tp0 * (sh_ref[...] * 0.25)
  out_ref[...] = jnp.concatenate(
      [tp, jnp.ones((BE, WIDTH - D_OUT), jnp.float32)], axis=1)


def _dense(ea, x1, sh, w1, b1, w2e):
  grid = (E // BE,)
  return pl.pallas_call(
      _dense_body,
      grid=grid,
      in_specs=[
          pl.BlockSpec((BE, D_EDGE), lambda i: (i, 0)),
          pl.BlockSpec((BE, D_IN), lambda i: (i, 0)),
          pl.BlockSpec((BE, 1), lambda i: (i, 0)),
          pl.BlockSpec((D_EDGE, HIDDEN), lambda i: (0, 0)),
          pl.BlockSpec((1, HIDDEN), lambda i: (0, 0)),
          pl.BlockSpec((HIDDEN * D_IN + D_IN, D_OUT), lambda i: (0, 0)),
      ],
      out_specs=pl.BlockSpec((BE, WIDTH), lambda i: (i, 0)),
      out_shape=jax.ShapeDtypeStruct((E, WIDTH), jnp.float32),
  )(ea, x1, sh, w1, b1, w2e)


BN = 400


def _combine_body(p0_ref, p1_ref, na_ref, out_ref):
  p0 = p0_ref[...]
  p1 = p1_ref[...]
  ssum = p0[:, :D_OUT] + p1[:, :D_OUT]
  cnt = p0[:, D_OUT:D_OUT + 1] + p1[:, D_OUT:D_OUT + 1]
  res = jnp.concatenate(
      [na_ref[...], jnp.zeros((BN, D_OUT - D_IN), jnp.float32)], axis=1)
  out_ref[...] = ssum / jnp.maximum(cnt, 1.0) + res


def _combine(p0, p1, na):
  grid = (N_NODES // BN,)
  return pl.pallas_call(
      _combine_body,
      grid=grid,
      in_specs=[
          pl.BlockSpec((BN, WIDTH), lambda i: (i, 0)),
          pl.BlockSpec((BN, WIDTH), lambda i: (i, 0)),
          pl.BlockSpec((BN, D_IN), lambda i: (i, 0)),
      ],
      out_specs=pl.BlockSpec((BN, D_OUT), lambda i: (i, 0)),
      out_shape=jax.ShapeDtypeStruct((N_NODES, D_OUT), jnp.float32),
  )(p0, p1, na)


@jax.jit
def kernel(node_attr, edge_index, edge_attr, edge_sh, W1, b1, W2, b2):
  edge_src = edge_index[0].astype(jnp.int32)
  edge_dst = edge_index[1].astype(jnp.int32)

  dst2 = edge_dst.reshape(NCHUNK, CHUNK)
  src2 = edge_src.reshape(NCHUNK, CHUNK)
  w2r = W2.reshape(HIDDEN, D_IN, D_OUT).reshape(HIDDEN * D_IN, D_OUT)
  b2r = b2.reshape(D_IN, D_OUT)
  w2e = jnp.concatenate([w2r, b2r], axis=0)
  b1r = b1.reshape(1, HIDDEN)
  zeros_np = jnp.zeros((N_PAD, WIDTH), jnp.float32)

  x1 = _gather(node_attr, dst2)
  tpc = _dense(edge_attr, x1, edge_sh, W1, b1r, w2e)
  partials = _scatter(tpc, src2, zeros_np)
  return _combine(partials[:N_NODES], partials[N_PAD:N_PAD + N_NODES],
                  node_attr)


# speedup vs baseline: 2.1189x; 2.1189x over previous
import jax
import jax.numpy as jnp
from jax import lax
from jax.experimental import pallas as pl
from jax.experimental.pallas import tpu as pltpu
import jax.experimental.pallas.tpu_sc as plsc

N_NODES = 10000
E = 160000
D_IN = 16
D_OUT = 32
D_EDGE = 16
HIDDEN = 16

NC = 2
NS = 16
NW = NC * NS
CHUNK = 128
NCHUNK = E // CHUNK
MAXCPW = -(-NCHUNK // NW)
WIDTH = 40
N_PAD = 10240
ROWS_PS = N_PAD // NS


def _sc_mesh():
  return plsc.VectorSubcoreMesh(
      core_axis_name="c", subcore_axis_name="s", num_cores=NC, num_subcores=NS)


def _worker_range(wid):
  c0 = (wid * NCHUNK) // NW
  c1 = ((wid + 1) * NCHUNK) // NW
  return c0, c1


def _gather_body(node_hbm, ei2_hbm, x1_hbm, idx_v, buf_v, sem):
  c = lax.axis_index("c")
  s = lax.axis_index("s")
  wid = s * NC + c
  c0, c1 = _worker_range(wid)
  t = c1 - c0
  pltpu.sync_copy(ei2_hbm.at[1, pl.ds(c0, MAXCPW)], idx_v)

  def fire(j, _):
    pltpu.async_copy(node_hbm.at[idx_v.at[j]],
                     buf_v.at[pl.ds(j * CHUNK, CHUNK)], sem)
    return _

  lax.fori_loop(0, t, fire, 0, unroll=False)
  base = c0 * CHUNK
  low = (MAXCPW - 1) * CHUNK
  pltpu.make_async_copy(x1_hbm.at[pl.ds(0, low)],
                        buf_v.at[pl.ds(0, low)], sem).wait()

  @pl.when(t == MAXCPW)
  def _():
    pltpu.make_async_copy(x1_hbm.at[pl.ds(0, CHUNK)],
                          buf_v.at[pl.ds(0, CHUNK)], sem).wait()

  pltpu.sync_copy(buf_v.at[pl.ds(0, low)], x1_hbm.at[pl.ds(base, low)])

  @pl.when(t == MAXCPW)
  def _():
    pltpu.sync_copy(buf_v.at[pl.ds(low, CHUNK)],
                    x1_hbm.at[pl.ds(base + low, CHUNK)])


def _gather(node_attr, ei2):
  return pl.kernel(
      _gather_body,
      out_type=jax.ShapeDtypeStruct((E, D_IN), jnp.float32),
      mesh=_sc_mesh(),
      scratch_types=[
          pltpu.VMEM((MAXCPW, CHUNK), jnp.int32),
          pltpu.VMEM((MAXCPW * CHUNK, D_IN), jnp.float32),
          pltpu.SemaphoreType.DMA,
      ],
      compiler_params=pltpu.CompilerParams(use_tc_tiling_on_sc=False),
  )(node_attr, ei2)


def _scatter_body(tpc_hbm, ei2_hbm, zeros_hbm, out_hbm, idx_v, val_v, acc_sh):
  c = lax.axis_index("c")
  s = lax.axis_index("s")
  wid = s * NC + c
  c0, c1 = _worker_range(wid)
  pltpu.sync_copy(ei2_hbm.at[0, pl.ds(c0, MAXCPW)], idx_v)
  pltpu.sync_copy(zeros_hbm.at[pl.ds(s * ROWS_PS, ROWS_PS)],
                  acc_sh.at[pl.ds(s * ROWS_PS, ROWS_PS)])
  plsc.subcore_barrier()

  def step(j, _):
    pltpu.sync_copy(tpc_hbm.at[pl.ds((c0 + j) * CHUNK, CHUNK)], val_v)
    pltpu.sync_copy(val_v, acc_sh.at[idx_v.at[j]], add=True)
    return _

  lax.fori_loop(0, c1 - c0, step, 0, unroll=False)
  plsc.subcore_barrier()
  pltpu.sync_copy(acc_sh.at[pl.ds(s * ROWS_PS, ROWS_PS)],
                  out_hbm.at[pl.ds(c * N_PAD + s * ROWS_PS, ROWS_PS)])


def _scatter(tpc, ei2, zeros_np):
  return pl.kernel(
      _scatter_body,
      out_type=jax.ShapeDtypeStruct((NC * N_PAD, WIDTH), jnp.float32),
      mesh=_sc_mesh(),
      scratch_types=[
          pltpu.VMEM((MAXCPW, CHUNK), jnp.int32),
          pltpu.VMEM((CHUNK, WIDTH), jnp.float32),
          pltpu.VMEM_SHARED((N_PAD, WIDTH), jnp.float32),
      ],
      compiler_params=pltpu.CompilerParams(use_tc_tiling_on_sc=False),
  )(tpc, ei2, zeros_np)


def _mish(x):
  sp = jnp.maximum(x, 0.0) + jnp.log1p(jnp.exp(-jnp.abs(x)))
  return x * jnp.tanh(sp)


BE = 3200


def _dense_body(ea_ref, x1_ref, sh_ref, w1_ref, b1_ref, rr_ref, tt_ref,
                w2e_ref, out_ref):
  ea = ea_ref[...]
  x1 = x1_ref[...]
  h = _mish(jnp.dot(ea, w1_ref[...], preferred_element_type=jnp.float32)
            + b1_ref[...])
  hrep = jnp.dot(h, rr_ref[...], preferred_element_type=jnp.float32)
  x1t = jnp.dot(x1, tt_ref[...], preferred_element_type=jnp.float32)
  ux = jnp.concatenate([hrep * x1t, x1], axis=1)
  tp0 = jnp.dot(ux, w2e_ref[...], preferred_element_type=jnp.float32)
  tp = tp0 * (sh_ref[...] * 0.25)
  out_ref[...] = jnp.concatenate(
      [tp, jnp.ones((BE, WIDTH - D_OUT), jnp.float32)], axis=1)


def _dense(ea, x1, sh, w1, b1, rr, tt, w2e):
  grid = (E // BE,)
  return pl.pallas_call(
      _dense_body,
      grid=grid,
      in_specs=[
          pl.BlockSpec((BE, D_EDGE), lambda i: (i, 0)),
          pl.BlockSpec((BE, D_IN), lambda i: (i, 0)),
          pl.BlockSpec((BE, 1), lambda i: (i, 0)),
          pl.BlockSpec((D_EDGE, HIDDEN), lambda i: (0, 0)),
          pl.BlockSpec((1, HIDDEN), lambda i: (0, 0)),
          pl.BlockSpec((HIDDEN, 256), lambda i: (0, 0)),
          pl.BlockSpec((D_IN, 256), lambda i: (0, 0)),
          pl.BlockSpec((HIDDEN * D_IN + D_IN, D_OUT), lambda i: (0, 0)),
      ],
      out_specs=pl.BlockSpec((BE, WIDTH), lambda i: (i, 0)),
      out_shape=jax.ShapeDtypeStruct((E, WIDTH), jnp.float32),
  )(ea, x1, sh, w1, b1, rr, tt, w2e)


BN = 400


def _combine_body(p0_ref, p1_ref, na_ref, out_ref):
  p0 = p0_ref[...]
  p1 = p1_ref[...]
  ssum = p0[:, :D_OUT] + p1[:, :D_OUT]
  cnt = p0[:, D_OUT:D_OUT + 1] + p1[:, D_OUT:D_OUT + 1]
  res = jnp.concatenate(
      [na_ref[...], jnp.zeros((BN, D_OUT - D_IN), jnp.float32)], axis=1)
  out_ref[...] = ssum / jnp.maximum(cnt, 1.0) + res


def _combine(p0, p1, na):
  grid = (N_NODES // BN,)
  return pl.pallas_call(
      _combine_body,
      grid=grid,
      in_specs=[
          pl.BlockSpec((BN, WIDTH), lambda i: (i, 0)),
          pl.BlockSpec((BN, WIDTH), lambda i: (i, 0)),
          pl.BlockSpec((BN, D_IN), lambda i: (i, 0)),
      ],
      out_specs=pl.BlockSpec((BN, D_OUT), lambda i: (i, 0)),
      out_shape=jax.ShapeDtypeStruct((N_NODES, D_OUT), jnp.float32),
  )(p0, p1, na)


@jax.jit
def kernel(node_attr, edge_index, edge_attr, edge_sh, W1, b1, W2, b2):
  ei2 = edge_index.astype(jnp.int32).reshape(2, NCHUNK, CHUNK)
  j = jnp.arange(256)
  rr = (j[None, :] // 16 == jnp.arange(HIDDEN)[:, None]).astype(jnp.float32)
  tt = (j[None, :] % 16 == jnp.arange(D_IN)[:, None]).astype(jnp.float32)
  w2r = W2.reshape(HIDDEN, D_IN, D_OUT).reshape(HIDDEN * D_IN, D_OUT)
  b2r = b2.reshape(D_IN, D_OUT)
  w2e = jnp.concatenate([w2r, b2r], axis=0)
  b1r = b1.reshape(1, HIDDEN)
  zeros_np = jnp.zeros((N_PAD, WIDTH), jnp.float32)

  x1 = _gather(node_attr, ei2)
  tpc = _dense(edge_attr, x1, edge_sh, W1, b1r, rr, tt, w2e)
  partials = _scatter(tpc, ei2, zeros_np)
  return _combine(partials[:N_NODES], partials[N_PAD:N_PAD + N_NODES],
                  node_attr)


# speedup vs baseline: 2.2423x; 1.0583x over previous
import jax
import jax.numpy as jnp
from jax import lax
from jax.experimental import pallas as pl
from jax.experimental.pallas import tpu as pltpu
import jax.experimental.pallas.tpu_sc as plsc

N_NODES = 10000
E = 160000
D_IN = 16
D_OUT = 32
D_EDGE = 16
HIDDEN = 16

NC = 2
NS = 16
NW = NC * NS
CHUNK = 128
NCHUNK = E // CHUNK
MAXCPW = -(-NCHUNK // NW)
WIDTH = 40
N_PAD = 10240
ROWS_PS = N_PAD // NS


def _sc_mesh():
  return plsc.VectorSubcoreMesh(
      core_axis_name="c", subcore_axis_name="s", num_cores=NC, num_subcores=NS)


def _worker_range(wid):
  c0 = (wid * NCHUNK) // NW
  c1 = ((wid + 1) * NCHUNK) // NW
  return c0, c1


def _gather_body(node_hbm, ei2_hbm, x1_hbm, idx_v, buf_v, sem):
  c = lax.axis_index("c")
  s = lax.axis_index("s")
  wid = s * NC + c
  c0, c1 = _worker_range(wid)
  t = c1 - c0
  pltpu.sync_copy(ei2_hbm.at[1, pl.ds(c0, MAXCPW)], idx_v)

  def fire(j, _):
    pltpu.async_copy(node_hbm.at[idx_v.at[j]],
                     buf_v.at[pl.ds(j * CHUNK, CHUNK)], sem)
    return _

  lax.fori_loop(0, t, fire, 0, unroll=False)
  base = c0 * CHUNK
  low = (MAXCPW - 1) * CHUNK
  pltpu.make_async_copy(x1_hbm.at[pl.ds(0, low)],
                        buf_v.at[pl.ds(0, low)], sem).wait()

  @pl.when(t == MAXCPW)
  def _():
    pltpu.make_async_copy(x1_hbm.at[pl.ds(0, CHUNK)],
                          buf_v.at[pl.ds(0, CHUNK)], sem).wait()

  pltpu.sync_copy(buf_v.at[pl.ds(0, low)], x1_hbm.at[pl.ds(base, low)])

  @pl.when(t == MAXCPW)
  def _():
    pltpu.sync_copy(buf_v.at[pl.ds(low, CHUNK)],
                    x1_hbm.at[pl.ds(base + low, CHUNK)])


def _gather(node_attr, ei2):
  return pl.kernel(
      _gather_body,
      out_type=jax.ShapeDtypeStruct((E, D_IN), jnp.float32),
      mesh=_sc_mesh(),
      scratch_types=[
          pltpu.VMEM((MAXCPW, CHUNK), jnp.int32),
          pltpu.VMEM((MAXCPW * CHUNK, D_IN), jnp.float32),
          pltpu.SemaphoreType.DMA,
      ],
      compiler_params=pltpu.CompilerParams(use_tc_tiling_on_sc=False),
  )(node_attr, ei2)


def _scatter_body(tpc_hbm, ei2_hbm, zeros_hbm, out_hbm, idx_v, val_v, acc_sh):
  c = lax.axis_index("c")
  s = lax.axis_index("s")
  wid = s * NC + c
  c0, c1 = _worker_range(wid)
  pltpu.sync_copy(ei2_hbm.at[0, pl.ds(c0, MAXCPW)], idx_v)
  pltpu.sync_copy(zeros_hbm.at[pl.ds(s * ROWS_PS, ROWS_PS)],
                  acc_sh.at[pl.ds(s * ROWS_PS, ROWS_PS)])
  plsc.subcore_barrier()

  def step(j, _):
    pltpu.sync_copy(tpc_hbm.at[pl.ds((c0 + j) * CHUNK, CHUNK)], val_v)
    pltpu.sync_copy(val_v, acc_sh.at[idx_v.at[j]], add=True)
    return _

  lax.fori_loop(0, c1 - c0, step, 0, unroll=False)
  plsc.subcore_barrier()
  pltpu.sync_copy(acc_sh.at[pl.ds(s * ROWS_PS, ROWS_PS)],
                  out_hbm.at[pl.ds(c * N_PAD + s * ROWS_PS, ROWS_PS)])


def _scatter(tpc, ei2, zeros_np):
  return pl.kernel(
      _scatter_body,
      out_type=jax.ShapeDtypeStruct((NC * N_PAD, WIDTH), jnp.float32),
      mesh=_sc_mesh(),
      scratch_types=[
          pltpu.VMEM((MAXCPW, CHUNK), jnp.int32),
          pltpu.VMEM((CHUNK, WIDTH), jnp.float32),
          pltpu.VMEM_SHARED((N_PAD, WIDTH), jnp.float32),
      ],
      compiler_params=pltpu.CompilerParams(use_tc_tiling_on_sc=False),
  )(tpc, ei2, zeros_np)


def _mish(x):
  s = 1.0 + jnp.exp(jnp.minimum(x, 20.0))
  r = s * s
  return x * (r - 1.0) / (r + 1.0)


BE = 8000


def _dense_body(ea_ref, x1_ref, sh_ref, w1_ref, b1_ref, rr_ref, tt_ref,
                w2e_ref, out_ref):
  ea = ea_ref[...]
  x1 = x1_ref[...]
  h = _mish(jnp.dot(ea, w1_ref[...], preferred_element_type=jnp.float32)
            + b1_ref[...])
  hrep = jnp.dot(h, rr_ref[...], preferred_element_type=jnp.float32)
  x1t = jnp.dot(x1, tt_ref[...], preferred_element_type=jnp.float32)
  ux = jnp.concatenate([hrep * x1t, x1], axis=1)
  tp0 = jnp.dot(ux, w2e_ref[...], preferred_element_type=jnp.float32)
  tp = tp0 * sh_ref[...]
  out_ref[...] = jnp.concatenate(
      [tp, jnp.ones((BE, WIDTH - D_OUT), jnp.float32)], axis=1)


def _dense(ea, x1, sh, w1, b1, rr, tt, w2e):
  grid = (E // BE,)
  return pl.pallas_call(
      _dense_body,
      grid=grid,
      in_specs=[
          pl.BlockSpec((BE, D_EDGE), lambda i: (i, 0)),
          pl.BlockSpec((BE, D_IN), lambda i: (i, 0)),
          pl.BlockSpec((BE, 1), lambda i: (i, 0)),
          pl.BlockSpec((D_EDGE, HIDDEN), lambda i: (0, 0)),
          pl.BlockSpec((1, HIDDEN), lambda i: (0, 0)),
          pl.BlockSpec((HIDDEN, 256), lambda i: (0, 0)),
          pl.BlockSpec((D_IN, 256), lambda i: (0, 0)),
          pl.BlockSpec((HIDDEN * D_IN + D_IN, D_OUT), lambda i: (0, 0)),
      ],
      out_specs=pl.BlockSpec((BE, WIDTH), lambda i: (i, 0)),
      out_shape=jax.ShapeDtypeStruct((E, WIDTH), jnp.float32),
  )(ea, x1, sh, w1, b1, rr, tt, w2e)


BN = 400


def _combine_body(p0_ref, p1_ref, na_ref, out_ref):
  p0 = p0_ref[...]
  p1 = p1_ref[...]
  ssum = p0[:, :D_OUT] + p1[:, :D_OUT]
  cnt = p0[:, D_OUT:D_OUT + 1] + p1[:, D_OUT:D_OUT + 1]
  res = jnp.concatenate(
      [na_ref[...], jnp.zeros((BN, D_OUT - D_IN), jnp.float32)], axis=1)
  out_ref[...] = ssum / jnp.maximum(cnt, 1.0) + res


def _combine(p0, p1, na):
  grid = (N_NODES // BN,)
  return pl.pallas_call(
      _combine_body,
      grid=grid,
      in_specs=[
          pl.BlockSpec((BN, WIDTH), lambda i: (i, 0)),
          pl.BlockSpec((BN, WIDTH), lambda i: (i, 0)),
          pl.BlockSpec((BN, D_IN), lambda i: (i, 0)),
      ],
      out_specs=pl.BlockSpec((BN, D_OUT), lambda i: (i, 0)),
      out_shape=jax.ShapeDtypeStruct((N_NODES, D_OUT), jnp.float32),
  )(p0, p1, na)


@jax.jit
def kernel(node_attr, edge_index, edge_attr, edge_sh, W1, b1, W2, b2):
  ei2 = edge_index.astype(jnp.int32).reshape(2, NCHUNK, CHUNK)
  j = jnp.arange(256)
  rr = (j[None, :] // 16 == jnp.arange(HIDDEN)[:, None]).astype(jnp.float32)
  tt = (j[None, :] % 16 == jnp.arange(D_IN)[:, None]).astype(jnp.float32)
  w2r = W2.reshape(HIDDEN, D_IN, D_OUT).reshape(HIDDEN * D_IN, D_OUT)
  b2r = b2.reshape(D_IN, D_OUT)
  w2e = jnp.concatenate([w2r, b2r], axis=0) * 0.25
  b1r = b1.reshape(1, HIDDEN)
  zeros_np = jnp.zeros((N_PAD, WIDTH), jnp.float32)

  x1 = _gather(node_attr, ei2)
  tpc = _dense(edge_attr, x1, edge_sh, W1, b1r, rr, tt, w2e)
  partials = _scatter(tpc, ei2, zeros_np)
  return _combine(partials[:N_NODES], partials[N_PAD:N_PAD + N_NODES],
                  node_attr)


# speedup vs baseline: 2.8045x; 1.2507x over previous
import jax
import jax.numpy as jnp
from jax import lax
from jax.experimental import pallas as pl
from jax.experimental.pallas import tpu as pltpu
import jax.experimental.pallas.tpu_sc as plsc

N_NODES = 10000
E = 160000
D_IN = 16
D_OUT = 32
D_EDGE = 16
HIDDEN = 16

NC = 2
NS = 16
NW = NC * NS
CHUNK = 128
NCHUNK = E // CHUNK
MAXCPW = -(-NCHUNK // NW)
WIDTH = 40
N_PAD = 10240
ROWS_PS = N_PAD // NS


def _sc_mesh():
  return plsc.VectorSubcoreMesh(
      core_axis_name="c", subcore_axis_name="s", num_cores=NC, num_subcores=NS)


def _worker_range(wid):
  c0 = (wid * NCHUNK) // NW
  c1 = ((wid + 1) * NCHUNK) // NW
  return c0, c1


def _gather_body(node_hbm, ei2_hbm, x1_hbm, idx_v, buf_v, sem):
  c = lax.axis_index("c")
  s = lax.axis_index("s")
  wid = s * NC + c
  c0, c1 = _worker_range(wid)
  t = c1 - c0
  pltpu.sync_copy(ei2_hbm.at[1, pl.ds(c0, MAXCPW)], idx_v)

  def fire(j, _):
    pltpu.async_copy(node_hbm.at[idx_v.at[j]],
                     buf_v.at[pl.ds(j * CHUNK, CHUNK)], sem)
    return _

  lax.fori_loop(0, t, fire, 0, unroll=False)
  base = c0 * CHUNK
  low = (MAXCPW - 1) * CHUNK
  pltpu.make_async_copy(x1_hbm.at[pl.ds(0, low)],
                        buf_v.at[pl.ds(0, low)], sem).wait()

  @pl.when(t == MAXCPW)
  def _():
    pltpu.make_async_copy(x1_hbm.at[pl.ds(0, CHUNK)],
                          buf_v.at[pl.ds(0, CHUNK)], sem).wait()

  pltpu.sync_copy(buf_v.at[pl.ds(0, low)], x1_hbm.at[pl.ds(base, low)])

  @pl.when(t == MAXCPW)
  def _():
    pltpu.sync_copy(buf_v.at[pl.ds(low, CHUNK)],
                    x1_hbm.at[pl.ds(base + low, CHUNK)])


def _gather(node_attr, ei2):
  return pl.kernel(
      _gather_body,
      out_type=jax.ShapeDtypeStruct((E, D_IN), jnp.float32),
      mesh=_sc_mesh(),
      scratch_types=[
          pltpu.VMEM((MAXCPW, CHUNK), jnp.int32),
          pltpu.VMEM((MAXCPW * CHUNK, D_IN), jnp.float32),
          pltpu.SemaphoreType.DMA,
      ],
      compiler_params=pltpu.CompilerParams(use_tc_tiling_on_sc=False),
  )(node_attr, ei2)


def _scatter_body(tpc_hbm, ei2_hbm, zeros_hbm, out_hbm, idx_v, val_v, acc_sh):
  c = lax.axis_index("c")
  s = lax.axis_index("s")
  wid = s * NC + c
  c0, c1 = _worker_range(wid)
  pltpu.sync_copy(ei2_hbm.at[0, pl.ds(c0, MAXCPW)], idx_v)
  pltpu.sync_copy(zeros_hbm.at[pl.ds(s * ROWS_PS, ROWS_PS)],
                  acc_sh.at[pl.ds(s * ROWS_PS, ROWS_PS)])
  plsc.subcore_barrier()

  def step(j, _):
    pltpu.sync_copy(tpc_hbm.at[pl.ds((c0 + j) * CHUNK, CHUNK)], val_v)
    pltpu.sync_copy(val_v, acc_sh.at[idx_v.at[j]], add=True)
    return _

  lax.fori_loop(0, c1 - c0, step, 0, unroll=False)
  plsc.subcore_barrier()
  pltpu.sync_copy(acc_sh.at[pl.ds(s * ROWS_PS, ROWS_PS)],
                  out_hbm.at[pl.ds(c * N_PAD + s * ROWS_PS, ROWS_PS)])


def _scatter(tpc, ei2, zeros_np):
  return pl.kernel(
      _scatter_body,
      out_type=jax.ShapeDtypeStruct((NC * N_PAD, WIDTH), jnp.float32),
      mesh=_sc_mesh(),
      scratch_types=[
          pltpu.VMEM((MAXCPW, CHUNK), jnp.int32),
          pltpu.VMEM((CHUNK, WIDTH), jnp.float32),
          pltpu.VMEM_SHARED((N_PAD, WIDTH), jnp.float32),
      ],
      compiler_params=pltpu.CompilerParams(use_tc_tiling_on_sc=False),
  )(tpc, ei2, zeros_np)


def _mish(x):
  s = 1.0 + jnp.exp(jnp.minimum(x, 20.0))
  r = s * s
  return x * (r - 1.0) / (r + 1.0)


BE = 6400


def _dense_body(eat_ref, x1_ref, sht_ref, w1t_ref, b1c_ref, rr_ref, tt_ref,
                w2e_ref, out_ref):
  ht = _mish(jnp.dot(w1t_ref[...], eat_ref[...],
                     preferred_element_type=jnp.float32) + b1c_ref[...])
  h = jnp.transpose(ht, (1, 0))
  x1 = x1_ref[...]
  hrep = jnp.dot(h, rr_ref[...], preferred_element_type=jnp.float32)
  x1t = jnp.dot(x1, tt_ref[...], preferred_element_type=jnp.float32)
  ux = jnp.concatenate([hrep * x1t, x1], axis=1)
  tp0 = jnp.dot(ux, w2e_ref[...], preferred_element_type=jnp.float32)
  tp = tp0 * jnp.transpose(sht_ref[...], (1, 0))
  out_ref[...] = jnp.concatenate(
      [tp, jnp.ones((BE, WIDTH - D_OUT), jnp.float32)], axis=1)


def _dense(eat, x1, sht, w1t, b1c, rr, tt, w2e):
  grid = (E // BE,)
  return pl.pallas_call(
      _dense_body,
      grid=grid,
      in_specs=[
          pl.BlockSpec((D_EDGE, BE), lambda i: (0, i)),
          pl.BlockSpec((BE, D_IN), lambda i: (i, 0)),
          pl.BlockSpec((1, BE), lambda i: (0, i)),
          pl.BlockSpec((HIDDEN, D_EDGE), lambda i: (0, 0)),
          pl.BlockSpec((HIDDEN, 1), lambda i: (0, 0)),
          pl.BlockSpec((HIDDEN, 256), lambda i: (0, 0)),
          pl.BlockSpec((D_IN, 256), lambda i: (0, 0)),
          pl.BlockSpec((HIDDEN * D_IN + D_IN, D_OUT), lambda i: (0, 0)),
      ],
      out_specs=pl.BlockSpec((BE, WIDTH), lambda i: (i, 0)),
      out_shape=jax.ShapeDtypeStruct((E, WIDTH), jnp.float32),
  )(eat, x1, sht, w1t, b1c, rr, tt, w2e)


BN = 400


def _combine_body(p0_ref, p1_ref, na_ref, out_ref):
  p0 = p0_ref[...]
  p1 = p1_ref[...]
  ssum = p0[:, :D_OUT] + p1[:, :D_OUT]
  cnt = p0[:, D_OUT:D_OUT + 1] + p1[:, D_OUT:D_OUT + 1]
  res = jnp.concatenate(
      [na_ref[...], jnp.zeros((BN, D_OUT - D_IN), jnp.float32)], axis=1)
  out_ref[...] = ssum / jnp.maximum(cnt, 1.0) + res


def _combine(p0, p1, na):
  grid = (N_NODES // BN,)
  return pl.pallas_call(
      _combine_body,
      grid=grid,
      in_specs=[
          pl.BlockSpec((BN, WIDTH), lambda i: (i, 0)),
          pl.BlockSpec((BN, WIDTH), lambda i: (i, 0)),
          pl.BlockSpec((BN, D_IN), lambda i: (i, 0)),
      ],
      out_specs=pl.BlockSpec((BN, D_OUT), lambda i: (i, 0)),
      out_shape=jax.ShapeDtypeStruct((N_NODES, D_OUT), jnp.float32),
  )(p0, p1, na)


@jax.jit
def kernel(node_attr, edge_index, edge_attr, edge_sh, W1, b1, W2, b2):
  ei2 = edge_index.astype(jnp.int32).reshape(2, NCHUNK, CHUNK)
  j = jnp.arange(256)
  rr = (j[None, :] // 16 == jnp.arange(HIDDEN)[:, None]).astype(jnp.float32)
  tt = (j[None, :] % 16 == jnp.arange(D_IN)[:, None]).astype(jnp.float32)
  w2r = W2.reshape(HIDDEN, D_IN, D_OUT).reshape(HIDDEN * D_IN, D_OUT)
  b2r = b2.reshape(D_IN, D_OUT)
  w2e = jnp.concatenate([w2r, b2r], axis=0) * 0.25
  b1c = b1.reshape(HIDDEN, 1)
  zeros_np = jnp.zeros((N_PAD, WIDTH), jnp.float32)

  x1 = _gather(node_attr, ei2)
  tpc = _dense(edge_attr.T, x1, edge_sh.T, W1.T, b1c, rr, tt, w2e)
  partials = _scatter(tpc, ei2, zeros_np)
  return _combine(partials[:N_NODES], partials[N_PAD:N_PAD + N_NODES],
                  node_attr)


# speedup vs baseline: 2.9652x; 1.0573x over previous
import jax
import jax.numpy as jnp
from jax import lax
from jax.experimental import pallas as pl
from jax.experimental.pallas import tpu as pltpu
import jax.experimental.pallas.tpu_sc as plsc

N_NODES = 10000
E = 160000
D_IN = 16
D_OUT = 32
D_EDGE = 16
HIDDEN = 16

NC = 2
NS = 16
NW = NC * NS
CHUNK = 128
NCHUNK = E // CHUNK
MAXCPW = -(-NCHUNK // NW)
WIDTH = 40
N_PAD = 10240
ROWS_PS = N_PAD // NS


def _sc_mesh():
  return plsc.VectorSubcoreMesh(
      core_axis_name="c", subcore_axis_name="s", num_cores=NC, num_subcores=NS)


def _worker_range(wid):
  c0 = (wid * NCHUNK) // NW
  c1 = ((wid + 1) * NCHUNK) // NW
  return c0, c1


def _gather_body(node_hbm, ei2_hbm, x1_hbm, idx_v, buf_v, sem):
  c = lax.axis_index("c")
  s = lax.axis_index("s")
  wid = s * NC + c
  c0, c1 = _worker_range(wid)
  t = c1 - c0
  pltpu.sync_copy(ei2_hbm.at[1, pl.ds(c0, MAXCPW)], idx_v)

  def fire(j, _):
    pltpu.async_copy(node_hbm.at[idx_v.at[j]],
                     buf_v.at[pl.ds(j * CHUNK, CHUNK)], sem)
    return _

  lax.fori_loop(0, t, fire, 0, unroll=False)
  base = c0 * CHUNK
  low = (MAXCPW - 1) * CHUNK
  pltpu.make_async_copy(x1_hbm.at[pl.ds(0, low)],
                        buf_v.at[pl.ds(0, low)], sem).wait()

  @pl.when(t == MAXCPW)
  def _():
    pltpu.make_async_copy(x1_hbm.at[pl.ds(0, CHUNK)],
                          buf_v.at[pl.ds(0, CHUNK)], sem).wait()

  pltpu.sync_copy(buf_v.at[pl.ds(0, low)], x1_hbm.at[pl.ds(base, low)])

  @pl.when(t == MAXCPW)
  def _():
    pltpu.sync_copy(buf_v.at[pl.ds(low, CHUNK)],
                    x1_hbm.at[pl.ds(base + low, CHUNK)])


def _gather(node_attr, ei2):
  return pl.kernel(
      _gather_body,
      out_type=jax.ShapeDtypeStruct((E, D_IN), jnp.float32),
      mesh=_sc_mesh(),
      scratch_types=[
          pltpu.VMEM((MAXCPW, CHUNK), jnp.int32),
          pltpu.VMEM((MAXCPW * CHUNK, D_IN), jnp.float32),
          pltpu.SemaphoreType.DMA,
      ],
      compiler_params=pltpu.CompilerParams(use_tc_tiling_on_sc=False),
  )(node_attr, ei2)


def _scatter_body(tpc_hbm, ei2_hbm, zeros_hbm, out_hbm, idx_v, val_v, acc_sh,
                  sem0, sem1):
  c = lax.axis_index("c")
  s = lax.axis_index("s")
  wid = s * NC + c
  c0, c1 = _worker_range(wid)
  t = c1 - c0
  pltpu.sync_copy(ei2_hbm.at[0, pl.ds(c0, MAXCPW)], idx_v)
  pltpu.sync_copy(zeros_hbm.at[pl.ds(s * ROWS_PS, ROWS_PS)],
                  acc_sh.at[pl.ds(s * ROWS_PS, ROWS_PS)])
  plsc.subcore_barrier()

  def load(j, slot, sem):
    pltpu.async_copy(tpc_hbm.at[pl.ds((c0 + j) * CHUNK, CHUNK)],
                     val_v.at[slot], sem)

  def wait(slot, sem):
    pltpu.make_async_copy(tpc_hbm.at[pl.ds(0, CHUNK)], val_v.at[slot],
                          sem).wait()

  load(0, 0, sem0)

  def step(j, carry):
    slot = lax.rem(j, 2)

    @pl.when(j + 1 < t)
    def _prefetch():
      @pl.when(slot == 0)
      def _p1():
        load(j + 1, 1, sem1)

      @pl.when(slot == 1)
      def _p0():
        load(j + 1, 0, sem0)

    @pl.when(slot == 0)
    def _s0():
      wait(0, sem0)
      pltpu.sync_copy(val_v.at[0], acc_sh.at[idx_v.at[j]], add=True)

    @pl.when(slot == 1)
    def _s1():
      wait(1, sem1)
      pltpu.sync_copy(val_v.at[1], acc_sh.at[idx_v.at[j]], add=True)

    return carry

  lax.fori_loop(0, t, step, 0, unroll=False)
  plsc.subcore_barrier()
  pltpu.sync_copy(acc_sh.at[pl.ds(s * ROWS_PS, ROWS_PS)],
                  out_hbm.at[pl.ds(c * N_PAD + s * ROWS_PS, ROWS_PS)])


def _scatter(tpc, ei2, zeros_np):
  return pl.kernel(
      _scatter_body,
      out_type=jax.ShapeDtypeStruct((NC * N_PAD, WIDTH), jnp.float32),
      mesh=_sc_mesh(),
      scratch_types=[
          pltpu.VMEM((MAXCPW, CHUNK), jnp.int32),
          pltpu.VMEM((2, CHUNK, WIDTH), jnp.float32),
          pltpu.VMEM_SHARED((N_PAD, WIDTH), jnp.float32),
          pltpu.SemaphoreType.DMA,
          pltpu.SemaphoreType.DMA,
      ],
      compiler_params=pltpu.CompilerParams(use_tc_tiling_on_sc=False),
  )(tpc, ei2, zeros_np)


def _mish(x):
  s = 1.0 + jnp.exp(jnp.minimum(x, 20.0))
  r = s * s
  return x * (r - 1.0) / (r + 1.0)


BE = 6400


def _dense_body(eat_ref, x1_ref, sht_ref, w1t_ref, b1c_ref, rr_ref, tt_ref,
                w2e_ref, out_ref):
  ht = _mish(jnp.dot(w1t_ref[...], eat_ref[...],
                     preferred_element_type=jnp.float32) + b1c_ref[...])
  h = jnp.transpose(ht, (1, 0))
  x1 = x1_ref[...]
  hrep = jnp.dot(h, rr_ref[...], preferred_element_type=jnp.float32)
  x1t = jnp.dot(x1, tt_ref[...], preferred_element_type=jnp.float32)
  ux = jnp.concatenate([hrep * x1t, x1], axis=1)
  tp0 = jnp.dot(ux, w2e_ref[...], preferred_element_type=jnp.float32)
  tp = tp0 * jnp.transpose(sht_ref[...], (1, 0))
  out_ref[...] = jnp.concatenate(
      [tp, jnp.ones((BE, WIDTH - D_OUT), jnp.float32)], axis=1)


def _dense(eat, x1, sht, w1t, b1c, rr, tt, w2e):
  grid = (E // BE,)
  return pl.pallas_call(
      _dense_body,
      grid=grid,
      in_specs=[
          pl.BlockSpec((D_EDGE, BE), lambda i: (0, i)),
          pl.BlockSpec((BE, D_IN), lambda i: (i, 0)),
          pl.BlockSpec((1, BE), lambda i: (0, i)),
          pl.BlockSpec((HIDDEN, D_EDGE), lambda i: (0, 0)),
          pl.BlockSpec((HIDDEN, 1), lambda i: (0, 0)),
          pl.BlockSpec((HIDDEN, 256), lambda i: (0, 0)),
          pl.BlockSpec((D_IN, 256), lambda i: (0, 0)),
          pl.BlockSpec((HIDDEN * D_IN + D_IN, D_OUT), lambda i: (0, 0)),
      ],
      out_specs=pl.BlockSpec((BE, WIDTH), lambda i: (i, 0)),
      out_shape=jax.ShapeDtypeStruct((E, WIDTH), jnp.float32),
  )(eat, x1, sht, w1t, b1c, rr, tt, w2e)


BN = 400


def _combine_body(p0_ref, p1_ref, na_ref, out_ref):
  p0 = p0_ref[...]
  p1 = p1_ref[...]
  ssum = p0[:, :D_OUT] + p1[:, :D_OUT]
  cnt = p0[:, D_OUT:D_OUT + 1] + p1[:, D_OUT:D_OUT + 1]
  res = jnp.concatenate(
      [na_ref[...], jnp.zeros((BN, D_OUT - D_IN), jnp.float32)], axis=1)
  out_ref[...] = ssum / jnp.maximum(cnt, 1.0) + res


def _combine(p0, p1, na):
  grid = (N_NODES // BN,)
  return pl.pallas_call(
      _combine_body,
      grid=grid,
      in_specs=[
          pl.BlockSpec((BN, WIDTH), lambda i: (i, 0)),
          pl.BlockSpec((BN, WIDTH), lambda i: (i, 0)),
          pl.BlockSpec((BN, D_IN), lambda i: (i, 0)),
      ],
      out_specs=pl.BlockSpec((BN, D_OUT), lambda i: (i, 0)),
      out_shape=jax.ShapeDtypeStruct((N_NODES, D_OUT), jnp.float32),
  )(p0, p1, na)


@jax.jit
def kernel(node_attr, edge_index, edge_attr, edge_sh, W1, b1, W2, b2):
  ei2 = edge_index.astype(jnp.int32).reshape(2, NCHUNK, CHUNK)
  j = jnp.arange(256)
  rr = (j[None, :] // 16 == jnp.arange(HIDDEN)[:, None]).astype(jnp.float32)
  tt = (j[None, :] % 16 == jnp.arange(D_IN)[:, None]).astype(jnp.float32)
  w2r = W2.reshape(HIDDEN, D_IN, D_OUT).reshape(HIDDEN * D_IN, D_OUT)
  b2r = b2.reshape(D_IN, D_OUT)
  w2e = jnp.concatenate([w2r, b2r], axis=0) * 0.25
  b1c = b1.reshape(HIDDEN, 1)
  zeros_np = jnp.zeros((N_PAD, WIDTH), jnp.float32)

  x1 = _gather(node_attr, ei2)
  tpc = _dense(edge_attr.T, x1, edge_sh.T, W1.T, b1c, rr, tt, w2e)
  partials = _scatter(tpc, ei2, zeros_np)
  return _combine(partials[:N_NODES], partials[N_PAD:N_PAD + N_NODES],
                  node_attr)
